# Initial kernel scaffold; baseline (speedup 1.0000x reference)
import jax
import jax.numpy as jnp
from jax.experimental import pallas as pl


def kernel(x, edge_index, edge_attr, W0, b0, Wc, bc, W_out, b_out):
    raise NotImplementedError("write your pallas kernel here")


# speedup vs baseline: 5.9888x; 5.9888x over previous
import functools

import jax
import jax.numpy as jnp
from jax import lax
from jax.experimental import pallas as pl
from jax.experimental.pallas import tpu as pltpu
from jax.experimental.pallas import tpu_sc as plsc

ALPHA = 0.1
BETA = 0.5
NC = 2
NS = 16
LANES = 16
CH = 128


def _sc_mesh():
    return plsc.VectorSubcoreMesh(
        core_axis_name="c", subcore_axis_name="s",
        num_cores=NC, num_subcores=NS)


_SC_PARAMS = pltpu.CompilerParams(needs_layout_passes=False)


def _make_deg_kernel(e_pad, n_pad):
    nw = NC * NS
    pt = e_pad // nw
    nit = pt // CH
    slc = n_pad // NS

    @functools.partial(
        pl.kernel,
        out_type=jax.ShapeDtypeStruct((NC * n_pad,), jnp.float32),
        mesh=_sc_mesh(),
        compiler_params=_SC_PARAMS,
        scratch_types=[
            pltpu.VMEM((CH,), jnp.int32),
            pltpu.VMEM((CH,), jnp.float32),
            pltpu.VMEM((slc,), jnp.float32),
            pltpu.VMEM_SHARED((n_pad,), jnp.float32),
        ],
    )
    def deg_kernel(col_hbm, w_hbm, out_hbm, col_v, w_v, zb, s_deg):
        c = lax.axis_index("c")
        s = lax.axis_index("s")
        wid = c * NS + s

        def zero_body(i, carry):
            zb[pl.ds(i * LANES, LANES)] = jnp.zeros((LANES,), jnp.float32)
            return carry
        lax.fori_loop(0, slc // LANES, zero_body, 0)
        pltpu.sync_copy(zb, s_deg.at[pl.ds(s * slc, slc)])
        plsc.subcore_barrier()

        def edge_body(it, carry):
            base = wid * pt + it * CH
            pltpu.sync_copy(col_hbm.at[pl.ds(base, CH)], col_v)
            pltpu.sync_copy(w_hbm.at[pl.ds(base, CH)], w_v)
            pltpu.sync_copy(w_v, s_deg.at[col_v], add=True)
            return carry
        lax.fori_loop(0, nit, edge_body, 0)
        plsc.subcore_barrier()
        pltpu.sync_copy(s_deg.at[pl.ds(s * slc, slc)],
                        out_hbm.at[pl.ds(c * n_pad + s * slc, slc)])

    return deg_kernel


def _make_norm_kernel(e_pad, n_pad):
    nw = NC * NS
    pt = e_pad // nw
    nit = pt // CH

    @functools.partial(
        pl.kernel,
        out_type=jax.ShapeDtypeStruct((e_pad,), jnp.float32),
        mesh=_sc_mesh(),
        compiler_params=_SC_PARAMS,
        scratch_types=[
            pltpu.VMEM((n_pad,), jnp.float32),
            pltpu.VMEM((CH,), jnp.int32),
            pltpu.VMEM((CH,), jnp.int32),
            pltpu.VMEM((CH,), jnp.float32),
            pltpu.VMEM((CH,), jnp.float32),
        ],
    )
    def norm_kernel(dinv_hbm, row_hbm, col_hbm, w_hbm, out_hbm,
                    dinv_v, row_v, col_v, w_v, nrm_v):
        c = lax.axis_index("c")
        s = lax.axis_index("s")
        wid = c * NS + s
        pltpu.sync_copy(dinv_hbm, dinv_v)

        def body(it, carry):
            base = wid * pt + it * CH
            pltpu.sync_copy(row_hbm.at[pl.ds(base, CH)], row_v)
            pltpu.sync_copy(col_hbm.at[pl.ds(base, CH)], col_v)
            pltpu.sync_copy(w_hbm.at[pl.ds(base, CH)], w_v)

            def grp(g, carry2):
                sl = pl.ds(g * LANES, LANES)
                dr = plsc.load_gather(dinv_v, [row_v[sl]])
                dc = plsc.load_gather(dinv_v, [col_v[sl]])
                nrm_v[sl] = dr * w_v[sl] * dc
                return carry2
            lax.fori_loop(0, CH // LANES, grp, 0)
            pltpu.sync_copy(nrm_v, out_hbm.at[pl.ds(base, CH)])
            return carry
        lax.fori_loop(0, nit, body, 0)

    return norm_kernel


def _make_layer_kernel(e_pad, n_pad, hid):
    nw = NC * NS
    pt = e_pad // nw
    nit = pt // CH
    slc = n_pad // NS
    nz = slc // CH
    kreg = hid // LANES

    @functools.partial(
        pl.kernel,
        out_type=jax.ShapeDtypeStruct((NC * n_pad, hid), jnp.float32),
        mesh=_sc_mesh(),
        compiler_params=_SC_PARAMS,
        scratch_types=[
            pltpu.VMEM((CH,), jnp.int32),
            pltpu.VMEM((CH,), jnp.int32),
            pltpu.VMEM((CH,), jnp.float32),
            pltpu.VMEM((CH, hid), jnp.float32),
            pltpu.VMEM_SHARED((n_pad, hid), jnp.float32),
            pltpu.SemaphoreType.DMA,
        ],
    )
    def layer_kernel(cur_hbm, row_hbm, col_hbm, nrm_hbm, out_hbm,
                     row_v, col_v, norm_v, rows_v, s_agg, sem):
        c = lax.axis_index("c")
        s = lax.axis_index("s")
        wid = c * NS + s

        def zrow(i, carry):
            for k in range(kreg):
                rows_v[i, pl.ds(k * LANES, LANES)] = jnp.zeros(
                    (LANES,), jnp.float32)
            return carry
        lax.fori_loop(0, CH, zrow, 0)
        for j in range(nz):
            pltpu.sync_copy(rows_v, s_agg.at[pl.ds(s * slc + j * CH, CH)])
        plsc.subcore_barrier()

        def edge_body(it, carry):
            base = wid * pt + it * CH
            pltpu.sync_copy(row_hbm.at[pl.ds(base, CH)], row_v)
            pltpu.sync_copy(col_hbm.at[pl.ds(base, CH)], col_v)
            pltpu.sync_copy(nrm_hbm.at[pl.ds(base, CH)], norm_v)
            pltpu.async_copy(cur_hbm.at[row_v], rows_v, sem).wait()

            def scale(e, carry2):
                nb = plsc.load_gather(
                    norm_v, [jnp.full((LANES,), e, jnp.int32)])
                for k in range(kreg):
                    sl = pl.ds(k * LANES, LANES)
                    rows_v[e, sl] = rows_v[e, sl] * nb
                return carry2
            lax.fori_loop(0, CH, scale, 0)
            pltpu.sync_copy(rows_v, s_agg.at[col_v], add=True)
            return carry
        lax.fori_loop(0, nit, edge_body, 0)
        plsc.subcore_barrier()
        pltpu.sync_copy(s_agg.at[pl.ds(s * slc, slc)],
                        out_hbm.at[pl.ds(c * n_pad + s * slc, slc)])

    return layer_kernel


def _tc_dinv(deg2):
    n_pad = deg2.shape[1]

    def body(deg_ref, dinv_ref):
        deg = deg_ref[0:1, :] + deg_ref[1:2, :]
        dinv_ref[...] = jnp.where(
            deg > 0, lax.rsqrt(jnp.maximum(deg, 1e-12)), 0.0)

    return pl.pallas_call(
        body,
        out_shape=jax.ShapeDtypeStruct((1, n_pad), jnp.float32),
    )(deg2)


def _tc_input(x_pad, W0, b0):
    n_pad, d_in = x_pad.shape
    hid = W0.shape[1]
    bn = 1280
    grid = (n_pad // bn,)

    def body(x_ref, w_ref, b_ref, h_ref):
        h = jnp.dot(x_ref[...], w_ref[...],
                    preferred_element_type=jnp.float32) + b_ref[...]
        h_ref[...] = jnp.maximum(h, 0.0)

    return pl.pallas_call(
        body,
        grid=grid,
        in_specs=[pl.BlockSpec((bn, d_in), lambda i: (i, 0)),
                  pl.BlockSpec((d_in, hid), lambda i: (0, 0)),
                  pl.BlockSpec((1, hid), lambda i: (0, 0))],
        out_specs=pl.BlockSpec((bn, hid), lambda i: (i, 0)),
        out_shape=jax.ShapeDtypeStruct((n_pad, hid), jnp.float32),
    )(x_pad, W0, b0.reshape(1, -1))


def _tc_dense(aggA, aggB, h0, cur, W, b):
    n_pad, hid = h0.shape
    bn = 1280
    grid = (n_pad // bn,)

    def body(a_ref, b2_ref, h0_ref, cur_ref, w_ref, bias_ref, o_ref):
        support = ((1.0 - ALPHA) * (a_ref[...] + b2_ref[...])
                   + ALPHA * h0_ref[...])
        out = ((1.0 - BETA) * support
               + BETA * jnp.dot(support, w_ref[...],
                                preferred_element_type=jnp.float32)
               + bias_ref[...])
        o_ref[...] = jnp.maximum(out, 0.0) + cur_ref[...]

    return pl.pallas_call(
        body,
        grid=grid,
        in_specs=[pl.BlockSpec((bn, hid), lambda i: (i, 0)),
                  pl.BlockSpec((bn, hid), lambda i: (i, 0)),
                  pl.BlockSpec((bn, hid), lambda i: (i, 0)),
                  pl.BlockSpec((bn, hid), lambda i: (i, 0)),
                  pl.BlockSpec((hid, hid), lambda i: (0, 0)),
                  pl.BlockSpec((1, hid), lambda i: (0, 0))],
        out_specs=pl.BlockSpec((bn, hid), lambda i: (i, 0)),
        out_shape=jax.ShapeDtypeStruct((n_pad, hid), jnp.float32),
    )(aggA, aggB, h0, cur, W, b.reshape(1, -1))


def _tc_logits(cur, Wp, bp):
    n_pad, hid = cur.shape
    oc = Wp.shape[1]
    bn = 1280
    grid = (n_pad // bn,)

    def body(c_ref, w_ref, b_ref, o_ref):
        logits = jnp.dot(c_ref[...], w_ref[...],
                         preferred_element_type=jnp.float32) + b_ref[...]
        m = jnp.max(logits, axis=1, keepdims=True)
        lse = m + jnp.log(jnp.sum(jnp.exp(logits - m), axis=1,
                                  keepdims=True))
        o_ref[...] = logits - lse

    return pl.pallas_call(
        body,
        grid=grid,
        in_specs=[pl.BlockSpec((bn, hid), lambda i: (i, 0)),
                  pl.BlockSpec((hid, oc), lambda i: (0, 0)),
                  pl.BlockSpec((1, oc), lambda i: (0, 0))],
        out_specs=pl.BlockSpec((bn, oc), lambda i: (i, 0)),
        out_shape=jax.ShapeDtypeStruct((n_pad, oc), jnp.float32),
    )(cur, Wp, bp)


def kernel(x, edge_index, edge_attr, W0, b0, Wc, bc, W_out, b_out):
    N, d_in = x.shape
    hid = W0.shape[1]
    L = Wc.shape[0]
    out_c = W_out.shape[1]
    E = edge_index.shape[1]

    nw = NC * NS
    unit_n = NS * CH
    n_pad = ((N + unit_n - 1) // unit_n) * unit_n
    e_f = E + N
    unit_e = nw * CH
    e_pad = ((e_f + unit_e - 1) // unit_e) * unit_e
    pad_e = e_pad - e_f

    row = edge_index[0].astype(jnp.int32)
    col = edge_index[1].astype(jnp.int32)
    loop_idx = jnp.arange(N, dtype=jnp.int32)
    zpad_i = jnp.zeros((pad_e,), jnp.int32)
    row_f = jnp.concatenate([row, loop_idx, zpad_i])
    col_f = jnp.concatenate([col, loop_idx, zpad_i])
    w_f = jnp.concatenate([edge_attr.astype(jnp.float32),
                           jnp.ones((N,), jnp.float32),
                           jnp.zeros((pad_e,), jnp.float32)])

    deg2 = _make_deg_kernel(e_pad, n_pad)(col_f, w_f)
    dinv = _tc_dinv(deg2.reshape(NC, n_pad)).reshape(n_pad)
    norm = _make_norm_kernel(e_pad, n_pad)(dinv, row_f, col_f, w_f)

    x_pad = jnp.pad(x, ((0, n_pad - N), (0, 0)))
    h0 = _tc_input(x_pad, W0, b0)

    layer_k = _make_layer_kernel(e_pad, n_pad, hid)
    cur = h0
    for l in range(L):
        agg2 = layer_k(cur, row_f, col_f, norm)
        cur = _tc_dense(agg2[:n_pad], agg2[n_pad:], h0, cur,
                        Wc[l], bc[l])

    pad_c = 128 - out_c
    Wp = jnp.pad(W_out, ((0, 0), (0, pad_c)))
    bp = jnp.pad(b_out, (0, pad_c), constant_values=-1e30)
    ls = _tc_logits(cur, Wp, bp.reshape(1, -1))
    return ls[:N, :out_c]
